# baseline (speedup 1.0000x reference)
import jax
import jax.numpy as jnp
import numpy as np
from jax import lax
from jax.experimental import pallas as pl
from jax.experimental.pallas import tpu as pltpu
from jax.experimental.pallas import tpu_sc as plsc

NUM_USERS = 100000
NUM_ITEMS = 100000
EMBED = 64
EW = EMBED // 2
B = 4096
HIST = 50
HIST_PAD = 56
HIST_COLS = 64

NC = 2
NS = 16
NW = NC * NS
NPW = B // NW

NBUF = 8

PERM = np.concatenate([np.arange(0, 16), np.arange(32, 48),
                       np.arange(16, 32), np.arange(48, 64)])


def _sc_body(u2e_hbm, v2e_hbm, nodes_hbm, tgt_hbm, hist_hbm,
             self_out, tgt_out, neigh_out,
             nodes_v, tgtidx_v, hist_v, self_v, tgtrows_v, neigh_v,
             rows_bufs, s0, s1, s2, s3, s4, s5, s6, s7,
             sem_self, sem_tgt, sem_hist):
    sems = [s0, s1, s2, s3, s4, s5, s6, s7]
    wid = lax.axis_index("s") * NC + lax.axis_index("c")
    base = wid * NPW

    pltpu.sync_copy(nodes_hbm.at[pl.ds(base, NPW)], nodes_v)
    pltpu.sync_copy(tgt_hbm.at[pl.ds(base, NPW)], tgtidx_v)

    c_self = pltpu.async_copy(u2e_hbm.at[nodes_v], self_v, sem_self)
    c_tgt = pltpu.async_copy(v2e_hbm.at[tgtidx_v], tgtrows_v, sem_tgt)
    pltpu.async_copy(hist_hbm.at[nodes_v], hist_v, sem_hist).wait()

    def idx_ref(n):
        nn = jnp.minimum(n, NPW - 1)
        return v2e_hbm.at[hist_v.at[nn, pl.ds(0, HIST_PAD)]]

    for k in range(NBUF):
        pltpu.async_copy(idx_ref(k), rows_bufs.at[k], sems[k])

    m = jnp.full((16,), -65536, jnp.int32)

    def group_body(g, carry):
        for k in range(NBUF):
            n = g * NBUF + k
            buf = rows_bufs.at[k]
            pltpu.make_async_copy(idx_ref(n), buf, sems[k]).wait()

            def acc_body(j, accs):
                a0, a1, a2, a3 = accs
                x0 = rows_bufs[k, j, pl.ds(0, 16)]
                x1 = rows_bufs[k, j, pl.ds(16, 16)]
                b0 = a0 + lax.bitcast_convert_type(x0 << 16, jnp.float32)
                b1 = a1 + lax.bitcast_convert_type(x0 & m, jnp.float32)
                b2 = a2 + lax.bitcast_convert_type(x1 << 16, jnp.float32)
                b3 = a3 + lax.bitcast_convert_type(x1 & m, jnp.float32)
                return (b0, b1, b2, b3)

            z = jnp.zeros((16,), jnp.float32)
            a0, a1, a2, a3 = lax.fori_loop(0, HIST, acc_body, (z, z, z, z))
            neigh_v[n, pl.ds(0, 16)] = a0
            neigh_v[n, pl.ds(16, 16)] = a1
            neigh_v[n, pl.ds(32, 16)] = a2
            neigh_v[n, pl.ds(48, 16)] = a3
            pltpu.async_copy(idx_ref(n + NBUF), buf, sems[k])
        return carry

    lax.fori_loop(0, NPW // NBUF, group_body, 0)
    for k in range(NBUF):
        pltpu.make_async_copy(idx_ref(0), rows_bufs.at[k], sems[k]).wait()

    c_self.wait()
    c_tgt.wait()
    pltpu.sync_copy(self_v, self_out.at[pl.ds(base, NPW)])
    pltpu.sync_copy(tgtrows_v, tgt_out.at[pl.ds(base, NPW)])
    pltpu.sync_copy(neigh_v, neigh_out.at[pl.ds(base, NPW)])


@jax.jit
def _sc_gather(u2e_i, v2e_i, nodes, nodes_target, histp):
    mesh = plsc.VectorSubcoreMesh(core_axis_name="c", subcore_axis_name="s")
    f32 = jnp.float32
    i32 = jnp.int32
    out_type = (jax.ShapeDtypeStruct((B, EW), i32),
                jax.ShapeDtypeStruct((B, EW), i32),
                jax.ShapeDtypeStruct((B, EMBED), f32))
    scratch = [
        pltpu.VMEM((NPW,), i32),
        pltpu.VMEM((NPW,), i32),
        pltpu.VMEM((NPW, HIST_COLS), i32),
        pltpu.VMEM((NPW, EW), i32),
        pltpu.VMEM((NPW, EW), i32),
        pltpu.VMEM((NPW, EMBED), f32),
        pltpu.VMEM((NBUF, HIST_PAD, EW), i32),
    ] + [pltpu.SemaphoreType.DMA] * (NBUF + 3)
    return pl.kernel(
        _sc_body,
        out_type=out_type,
        mesh=mesh,
        scratch_types=scratch,
        compiler_params=pltpu.CompilerParams(use_tc_tiling_on_sc=False),
    )(u2e_i, v2e_i, nodes, nodes_target, histp)


def _tc_body(self_ref, tgt_ref, neigh_ref, w1_ref, wbp_ref, b1_ref, out_ref):
    w = w1_ref[...]
    wbp = wbp_ref[...]
    b = b1_ref[...]

    def unpack(x):
        lo = lax.bitcast_convert_type(x << 16, jnp.float32)
        hi = lax.bitcast_convert_type(x & jnp.int32(-65536), jnp.float32)
        return jnp.concatenate([lo, hi], axis=1)

    cf = jnp.concatenate([unpack(self_ref[...]), unpack(tgt_ref[...])], axis=1)
    h1 = lax.dot_general(cf, w, (((1,), (1,)), ((), ())),
                         preferred_element_type=jnp.float32)
    h1 = jnp.maximum(h1 + b, 0.0)
    neigh = neigh_ref[...] * (1.0 / HIST)
    h2 = (lax.dot_general(h1, w[:, :EMBED], (((1,), (1,)), ((), ())),
                          preferred_element_type=jnp.float32)
          + lax.dot_general(neigh, wbp, (((1,), (1,)), ((), ())),
                            preferred_element_type=jnp.float32))
    out_ref[...] = jnp.maximum(h2 + b, 0.0)


@jax.jit
def _tc_mlp(self_i, tgt_i, neigh_sum, W1, Wbp, b1):
    return pl.pallas_call(
        _tc_body,
        out_shape=jax.ShapeDtypeStruct((B, EMBED), jnp.float32),
    )(self_i, tgt_i, neigh_sum, W1, Wbp, b1.reshape(1, EMBED))


def kernel(u2e, v2e, W1, b1, nodes, nodes_target, history_u, uv):
    del uv
    nodes = nodes.astype(jnp.int32)
    nodes_target = nodes_target.astype(jnp.int32)
    history_u = history_u.astype(jnp.int32)
    histp = jnp.pad(history_u, ((0, 0), (0, HIST_COLS - HIST)))
    def pack(t):
        u = lax.bitcast_convert_type(t, jnp.int32)
        r = lax.shift_right_logical(
            u + 32767 + (lax.shift_right_logical(u, 16) & 1), 16)
        return r[:, :EW] | (r[:, EW:] << 16)

    u2e_i = pack(u2e)
    v2e_i = pack(v2e)
    self_i, tgt_i, neigh_sum = _sc_gather(
        u2e_i, v2e_i, nodes, nodes_target, histp)
    Wbp = W1[:, EMBED + PERM]
    return _tc_mlp(self_i, tgt_i, neigh_sum, W1, Wbp, b1)


# speedup vs baseline: 1.2031x; 1.2031x over previous
import jax
import jax.numpy as jnp
import numpy as np
from jax import lax
from jax.experimental import pallas as pl
from jax.experimental.pallas import tpu as pltpu
from jax.experimental.pallas import tpu_sc as plsc

NUM_USERS = 100000
NUM_ITEMS = 100000
EMBED = 64
EW = EMBED // 2
B = 4096
HIST = 50
HIST_PAD = 56
HIST_COLS = 64

NC = 2
NS = 16
NW = NC * NS
NPW = B // NW

NBUF = 8

PERM = np.concatenate([np.arange(0, 16), np.arange(32, 48),
                       np.arange(16, 32), np.arange(48, 64)])


def _sc_body(v2e_hbm, nodes_hbm, tgt_hbm, hist_hbm,
             tgt_out, neigh_out,
             nodes_v, tgtidx_v, hist_v, tgtrows_v, neigh_v,
             rows_bufs, s0, s1, s2, s3, s4, s5, s6, s7,
             sem_tgt, sem_hist):
    sems = [s0, s1, s2, s3, s4, s5, s6, s7]
    wid = lax.axis_index("s") * NC + lax.axis_index("c")
    base = wid * NPW

    pltpu.sync_copy(nodes_hbm.at[pl.ds(base, NPW)], nodes_v)
    pltpu.sync_copy(tgt_hbm.at[pl.ds(base, NPW)], tgtidx_v)

    c_tgt = pltpu.async_copy(v2e_hbm.at[tgtidx_v], tgtrows_v, sem_tgt)
    pltpu.async_copy(hist_hbm.at[nodes_v], hist_v, sem_hist).wait()

    def idx_ref(n):
        nn = jnp.minimum(n, NPW - 1)
        return v2e_hbm.at[hist_v.at[nn, pl.ds(0, HIST_PAD)]]

    for k in range(NBUF):
        pltpu.async_copy(idx_ref(k), rows_bufs.at[k], sems[k])

    m = jnp.full((16,), -65536, jnp.int32)

    def group_body(g, carry):
        for k in range(NBUF):
            n = g * NBUF + k
            buf = rows_bufs.at[k]
            pltpu.make_async_copy(idx_ref(n), buf, sems[k]).wait()

            def acc_body(j, accs):
                a0, a1, a2, a3 = accs
                x0 = rows_bufs[k, j, pl.ds(0, 16)]
                x1 = rows_bufs[k, j, pl.ds(16, 16)]
                b0 = a0 + lax.bitcast_convert_type(x0 << 16, jnp.float32)
                b1 = a1 + lax.bitcast_convert_type(x0 & m, jnp.float32)
                b2 = a2 + lax.bitcast_convert_type(x1 << 16, jnp.float32)
                b3 = a3 + lax.bitcast_convert_type(x1 & m, jnp.float32)
                return (b0, b1, b2, b3)

            z = jnp.zeros((16,), jnp.float32)
            a0, a1, a2, a3 = lax.fori_loop(0, HIST, acc_body, (z, z, z, z))
            neigh_v[n, pl.ds(0, 16)] = a0
            neigh_v[n, pl.ds(16, 16)] = a1
            neigh_v[n, pl.ds(32, 16)] = a2
            neigh_v[n, pl.ds(48, 16)] = a3
            pltpu.async_copy(idx_ref(n + NBUF), buf, sems[k])
        return carry

    lax.fori_loop(0, NPW // NBUF, group_body, 0)
    for k in range(NBUF):
        pltpu.make_async_copy(idx_ref(0), rows_bufs.at[k], sems[k]).wait()

    c_tgt.wait()
    pltpu.sync_copy(tgtrows_v, tgt_out.at[pl.ds(base, NPW)])
    pltpu.sync_copy(neigh_v, neigh_out.at[pl.ds(base, NPW)])


def _sc_self_body(u2e_hbm, nodes_hbm, self_out, nodes_v, self_v, sem):
    wid = lax.axis_index("s") * NC + lax.axis_index("c")
    base = wid * NPW
    pltpu.sync_copy(nodes_hbm.at[pl.ds(base, NPW)], nodes_v)
    pltpu.async_copy(u2e_hbm.at[nodes_v], self_v, sem).wait()
    pltpu.sync_copy(self_v, self_out.at[pl.ds(base, NPW)])


@jax.jit
def _sc_gather(v2e_i, nodes, nodes_target, histp):
    mesh = plsc.VectorSubcoreMesh(core_axis_name="c", subcore_axis_name="s")
    f32 = jnp.float32
    i32 = jnp.int32
    out_type = (jax.ShapeDtypeStruct((B, EW), i32),
                jax.ShapeDtypeStruct((B, EMBED), f32))
    scratch = [
        pltpu.VMEM((NPW,), i32),
        pltpu.VMEM((NPW,), i32),
        pltpu.VMEM((NPW, HIST_COLS), i32),
        pltpu.VMEM((NPW, EW), i32),
        pltpu.VMEM((NPW, EMBED), f32),
        pltpu.VMEM((NBUF, HIST_PAD, EW), i32),
    ] + [pltpu.SemaphoreType.DMA] * (NBUF + 2)
    return pl.kernel(
        _sc_body,
        out_type=out_type,
        mesh=mesh,
        scratch_types=scratch,
        compiler_params=pltpu.CompilerParams(use_tc_tiling_on_sc=False),
    )(v2e_i, nodes, nodes_target, histp)


@jax.jit
def _sc_gather_self(u2e_i, nodes):
    mesh = plsc.VectorSubcoreMesh(core_axis_name="c", subcore_axis_name="s")
    i32 = jnp.int32
    return pl.kernel(
        _sc_self_body,
        out_type=jax.ShapeDtypeStruct((B, EW), i32),
        mesh=mesh,
        scratch_types=[pltpu.VMEM((NPW,), i32),
                       pltpu.VMEM((NPW, EW), i32),
                       pltpu.SemaphoreType.DMA],
        compiler_params=pltpu.CompilerParams(use_tc_tiling_on_sc=False),
    )(u2e_i, nodes)


def _tc_body(self_ref, tgt_ref, neigh_ref, w1_ref, wbp_ref, b1_ref, out_ref):
    w = w1_ref[...]
    wbp = wbp_ref[...]
    b = b1_ref[...]

    def unpack(x):
        lo = lax.bitcast_convert_type(x << 16, jnp.float32)
        hi = lax.bitcast_convert_type(x & jnp.int32(-65536), jnp.float32)
        return jnp.concatenate([lo, hi], axis=1)

    cf = jnp.concatenate([unpack(self_ref[...]), unpack(tgt_ref[...])], axis=1)
    h1 = lax.dot_general(cf, w, (((1,), (1,)), ((), ())),
                         preferred_element_type=jnp.float32)
    h1 = jnp.maximum(h1 + b, 0.0)
    neigh = neigh_ref[...] * (1.0 / HIST)
    h2 = (lax.dot_general(h1, w[:, :EMBED], (((1,), (1,)), ((), ())),
                          preferred_element_type=jnp.float32)
          + lax.dot_general(neigh, wbp, (((1,), (1,)), ((), ())),
                            preferred_element_type=jnp.float32))
    out_ref[...] = jnp.maximum(h2 + b, 0.0)


@jax.jit
def _tc_mlp(self_i, tgt_i, neigh_sum, W1, Wbp, b1):
    return pl.pallas_call(
        _tc_body,
        out_shape=jax.ShapeDtypeStruct((B, EMBED), jnp.float32),
    )(self_i, tgt_i, neigh_sum, W1, Wbp, b1.reshape(1, EMBED))


def kernel(u2e, v2e, W1, b1, nodes, nodes_target, history_u, uv):
    del uv
    nodes = nodes.astype(jnp.int32)
    nodes_target = nodes_target.astype(jnp.int32)
    history_u = history_u.astype(jnp.int32)
    histp = jnp.pad(history_u, ((0, 0), (0, HIST_COLS - HIST)))
    def pack(t):
        tb = t.astype(jnp.bfloat16)
        lo = lax.bitcast_convert_type(tb[:, :EW], jnp.uint16).astype(jnp.uint32)
        hi = lax.bitcast_convert_type(tb[:, EW:], jnp.uint16).astype(jnp.uint32)
        return lax.bitcast_convert_type(lo | (hi << 16), jnp.int32)

    v2e_i = pack(v2e)
    tgt_i, neigh_sum = _sc_gather(v2e_i, nodes, nodes_target, histp)
    u2e_i = pack(u2e)
    self_i = _sc_gather_self(u2e_i, nodes)
    Wbp = W1[:, EMBED + PERM]
    return _tc_mlp(self_i, tgt_i, neigh_sum, W1, Wbp, b1)
